# baseline (speedup 1.0000x reference)
import jax
import jax.numpy as jnp
from jax.experimental import pallas as pl
from jax.experimental.pallas import tpu as pltpu

_N = 4096
_R = 256
_CH = 8
_LANES = 128
_NS = _N // _LANES
_INF = float("inf")

_BATCHER8 = [(0, 1), (2, 3), (4, 5), (6, 7),
             (0, 2), (1, 3), (1, 2),
             (4, 6), (5, 7), (5, 6),
             (0, 4), (1, 5), (2, 6), (3, 7),
             (2, 4), (3, 5),
             (1, 2), (3, 4), (5, 6)]
_BITONIC8 = [(0, 4), (1, 5), (2, 6), (3, 7),
             (0, 2), (1, 3), (4, 6), (5, 7),
             (0, 1), (2, 3), (4, 5), (6, 7)]


def _dist_block(rows, alln, d_ref):
    g = jax.lax.dot_general(rows, alln, (((1,), (1,)), ((), ())),
                            preferred_element_type=jnp.float32)
    sq_r = jnp.sum(rows * rows, axis=1, keepdims=True)
    ones = jnp.ones((1, rows.shape[1]), jnp.float32)
    sq_a = jax.lax.dot_general(ones, alln * alln, (((1,), (1,)), ((), ())),
                               preferred_element_type=jnp.float32)
    d_ref[...] = (sq_r + sq_a) - 2.0 * g


def _ce(a, i, j):
    lo = jnp.minimum(a[i], a[j])
    hi = jnp.maximum(a[i], a[j])
    a[i], a[j] = lo, hi


def _merge8(a, b):
    m = [jnp.minimum(a[i], b[7 - i]) for i in range(8)]
    for i, j in _BITONIC8:
        _ce(m, i, j)
    return m


def _top8_chunk(v):
    groups = []
    for g in range(4):
        a = list(v[8 * g:8 * g + 8])
        for i, j in _BATCHER8:
            _ce(a, i, j)
        groups.append(a)
    f = _merge8(_merge8(groups[0], groups[1]),
                _merge8(groups[2], groups[3]))
    for _ in range(7):
        t = jnp.min(f[0], axis=1, keepdims=True)
        pop = f[0] == t
        f = ([jnp.where(pop, f[k + 1], f[k]) for k in range(7)]
             + [jnp.where(pop, _INF, f[7])])
    return jnp.min(f[0], axis=1, keepdims=True)


def _chunk_slices(d_ref, r0):
    rows = pl.ds(r0, _CH)
    return [d_ref[rows, _LANES * k:_LANES * (k + 1)] for k in range(_NS)]


def _body(xr, xa, zr, za, out_ref, dx_ref, dz_ref):
    i = pl.program_id(0)

    _dist_block(xr[...], xa[...], dx_ref)
    _dist_block(zr[...], za[...], dz_ref)

    eye = (jax.lax.broadcasted_iota(jnp.int32, (_R, _R), 0)
           == jax.lax.broadcasted_iota(jnp.int32, (_R, _R), 1))
    win = pl.ds(i * _R, _R)
    dx_ref[:, win] = jnp.where(eye, _INF, dx_ref[:, win])
    dz_ref[:, win] = jnp.where(eye, _INF, dz_ref[:, win])

    def chunk(c, acc):
        r0 = c * _CH
        t8x = _top8_chunk(_chunk_slices(dx_ref, r0))
        t8z = _top8_chunk(_chunk_slices(dz_ref, r0))
        vx = _chunk_slices(dx_ref, r0)
        vz = _chunk_slices(dz_ref, r0)
        for k in range(_NS):
            both = (vx[k] <= t8x) & (vz[k] <= t8z)
            acc = acc + jnp.where(both, 1.0, 0.0)
        return acc

    acc = jax.lax.fori_loop(0, _R // _CH, chunk,
                            jnp.zeros((_CH, _LANES), jnp.float32))
    c = jnp.sum(acc)
    out_ref[...] = jnp.full((8, 128), (16.0 * _R - 2.0 * c) * (100.0 / (_N * _N)),
                            jnp.float32)


def kernel(X, Z):
    n, dx = X.shape
    _, dz = Z.shape
    out = pl.pallas_call(
        _body,
        grid=(n // _R,),
        in_specs=[
            pl.BlockSpec((_R, dx), lambda i: (i, 0)),
            pl.BlockSpec((n, dx), lambda i: (0, 0)),
            pl.BlockSpec((_R, dz), lambda i: (i, 0)),
            pl.BlockSpec((n, dz), lambda i: (0, 0)),
        ],
        out_specs=pl.BlockSpec((8, 128), lambda i: (i, 0)),
        out_shape=jax.ShapeDtypeStruct((8 * (n // _R), 128), jnp.float32),
        scratch_shapes=[
            pltpu.VMEM((_R, _N), jnp.float32),
            pltpu.VMEM((_R, _N), jnp.float32),
        ],
        compiler_params=pltpu.CompilerParams(
            dimension_semantics=("parallel",)),
    )(X, X, Z, Z)
    return jnp.sum(out[::8, 0])


# speedup vs baseline: 3.5473x; 3.5473x over previous
import jax
import jax.numpy as jnp
from jax.experimental import pallas as pl
from jax.experimental.pallas import tpu as pltpu

_N = 4096
_R = 256
_CH = 8
_LANES = 128
_NS = _N // _LANES
_INF = float("inf")

_BATCHER8 = [(0, 1), (2, 3), (4, 5), (6, 7),
             (0, 2), (1, 3), (1, 2),
             (4, 6), (5, 7), (5, 6),
             (0, 4), (1, 5), (2, 6), (3, 7),
             (2, 4), (3, 5),
             (1, 2), (3, 4), (5, 6)]
_BITONIC8 = [(0, 4), (1, 5), (2, 6), (3, 7),
             (0, 2), (1, 3), (4, 6), (5, 7),
             (0, 1), (2, 3), (4, 5), (6, 7)]


def _dist_block(rows, alln, d_ref):
    g = jax.lax.dot_general(rows, alln, (((1,), (1,)), ((), ())),
                            preferred_element_type=jnp.float32)
    sq_r = jnp.sum(rows * rows, axis=1, keepdims=True)
    ones = jnp.ones((1, rows.shape[1]), jnp.float32)
    sq_a = jax.lax.dot_general(ones, alln * alln, (((1,), (1,)), ((), ())),
                               preferred_element_type=jnp.float32)
    d_ref[...] = (sq_r + sq_a) - 2.0 * g


def _ce(a, i, j):
    lo = jnp.minimum(a[i], a[j])
    hi = jnp.maximum(a[i], a[j])
    a[i], a[j] = lo, hi


def _merge8(a, b):
    m = [jnp.minimum(a[i], b[7 - i]) for i in range(8)]
    for i, j in _BITONIC8:
        _ce(m, i, j)
    return m


def _top8_chunk(v):
    groups = []
    for g in range(4):
        a = list(v[8 * g:8 * g + 8])
        for i, j in _BATCHER8:
            _ce(a, i, j)
        groups.append(a)
    f = _merge8(_merge8(groups[0], groups[1]),
                _merge8(groups[2], groups[3]))
    for _ in range(7):
        t = jnp.min(f[0], axis=1, keepdims=True)
        pop = f[0] == t
        f = ([jnp.where(pop, f[k + 1], f[k]) for k in range(7)]
             + [jnp.where(pop, _INF, f[7])])
    return jnp.min(f[0], axis=1, keepdims=True)


def _chunk_slices(d_ref, r0):
    rows = pl.ds(r0, _CH)
    return [d_ref[rows, _LANES * k:_LANES * (k + 1)] for k in range(_NS)]


def _body(xr, xa, zr, za, out_ref, dx_ref, dz_ref):
    i = pl.program_id(0)

    _dist_block(xr[...], xa[...], dx_ref)
    _dist_block(zr[...], za[...], dz_ref)

    eye = (jax.lax.broadcasted_iota(jnp.int32, (_R, _R), 0)
           == jax.lax.broadcasted_iota(jnp.int32, (_R, _R), 1))
    win = pl.ds(i * _R, _R)
    dx_ref[:, win] = jnp.where(eye, _INF, dx_ref[:, win])
    dz_ref[:, win] = jnp.where(eye, _INF, dz_ref[:, win])

    t8x = jnp.concatenate(
        [_top8_chunk(_chunk_slices(dx_ref, _CH * c)) for c in range(_R // _CH)],
        axis=0)
    t8z = jnp.concatenate(
        [_top8_chunk(_chunk_slices(dz_ref, _CH * c)) for c in range(_R // _CH)],
        axis=0)

    both = (dx_ref[...] <= t8x) & (dz_ref[...] <= t8z)
    c = jnp.sum(both.astype(jnp.float32))
    out_ref[...] = jnp.full((8, 128), (16.0 * _R - 2.0 * c) * (100.0 / (_N * _N)),
                            jnp.float32)


def kernel(X, Z):
    n, dx = X.shape
    _, dz = Z.shape
    out = pl.pallas_call(
        _body,
        grid=(n // _R,),
        in_specs=[
            pl.BlockSpec((_R, dx), lambda i: (i, 0)),
            pl.BlockSpec((n, dx), lambda i: (0, 0)),
            pl.BlockSpec((_R, dz), lambda i: (i, 0)),
            pl.BlockSpec((n, dz), lambda i: (0, 0)),
        ],
        out_specs=pl.BlockSpec((8, 128), lambda i: (i, 0)),
        out_shape=jax.ShapeDtypeStruct((8 * (n // _R), 128), jnp.float32),
        scratch_shapes=[
            pltpu.VMEM((_R, _N), jnp.float32),
            pltpu.VMEM((_R, _N), jnp.float32),
        ],
        compiler_params=pltpu.CompilerParams(
            dimension_semantics=("parallel",)),
    )(X, X, Z, Z)
    return jnp.sum(out[::8, 0])


# speedup vs baseline: 4.7287x; 1.3330x over previous
import jax
import jax.numpy as jnp
from jax.experimental import pallas as pl
from jax.experimental.pallas import tpu as pltpu

_N = 4096
_R = 256
_CH = 8
_LANES = 128
_NS = _N // _LANES
_INF = float("inf")

_BATCHER8 = [(0, 1), (2, 3), (4, 5), (6, 7),
             (0, 2), (1, 3), (1, 2),
             (4, 6), (5, 7), (5, 6),
             (0, 4), (1, 5), (2, 6), (3, 7),
             (2, 4), (3, 5),
             (1, 2), (3, 4), (5, 6)]
_BITONIC8 = [(0, 4), (1, 5), (2, 6), (3, 7),
             (0, 2), (1, 3), (4, 6), (5, 7),
             (0, 1), (2, 3), (4, 5), (6, 7)]


def _sq_all(alln):
    ones = jnp.ones((1, alln.shape[1]), jnp.float32)
    return jax.lax.dot_general(ones, alln * alln, (((1,), (1,)), ((), ())),
                               preferred_element_type=jnp.float32)


def _dist_block(rows, alln, sq_a, d_ref):
    g2 = jax.lax.dot_general(-2.0 * rows, alln, (((1,), (1,)), ((), ())),
                             preferred_element_type=jnp.float32)
    sq_r = jnp.sum(rows * rows, axis=1, keepdims=True)
    d_ref[...] = (sq_r + sq_a) + g2


def _ce(a, i, j):
    lo = jnp.minimum(a[i], a[j])
    hi = jnp.maximum(a[i], a[j])
    a[i], a[j] = lo, hi


def _merge8(a, b):
    m = [jnp.minimum(a[i], b[7 - i]) for i in range(8)]
    for i, j in _BITONIC8:
        _ce(m, i, j)
    return m


def _top8_chunk(v):
    groups = []
    for g in range(4):
        a = list(v[8 * g:8 * g + 8])
        for i, j in _BATCHER8:
            _ce(a, i, j)
        groups.append(a)
    f = _merge8(_merge8(groups[0], groups[1]),
                _merge8(groups[2], groups[3]))
    for _ in range(7):
        t = jnp.min(f[0], axis=1, keepdims=True)
        pop = f[0] == t
        f = ([jnp.where(pop, f[k + 1], f[k]) for k in range(7)]
             + [jnp.where(pop, _INF, f[7])])
    return jnp.min(f[0], axis=1, keepdims=True)


def _chunk_slices(d_ref, r0):
    rows = pl.ds(r0, _CH)
    return [d_ref[rows, _LANES * k:_LANES * (k + 1)] for k in range(_NS)]


def _body(xr, xa, zr, za, out_ref, dx_ref, dz_ref, sqx_ref, sqz_ref):
    i = pl.program_id(0)

    @pl.when(i == 0)
    def _():
        sqx_ref[...] = _sq_all(xa[...])
        sqz_ref[...] = _sq_all(za[...])

    _dist_block(xr[...], xa[...], sqx_ref[...], dx_ref)
    _dist_block(zr[...], za[...], sqz_ref[...], dz_ref)

    eye = (jax.lax.broadcasted_iota(jnp.int32, (_R, _R), 0)
           == jax.lax.broadcasted_iota(jnp.int32, (_R, _R), 1))
    win = pl.ds(i * _R, _R)
    dx_ref[:, win] = jnp.where(eye, _INF, dx_ref[:, win])
    dz_ref[:, win] = jnp.where(eye, _INF, dz_ref[:, win])

    t8x = jnp.concatenate(
        [_top8_chunk(_chunk_slices(dx_ref, _CH * c)) for c in range(_R // _CH)],
        axis=0)
    t8z = jnp.concatenate(
        [_top8_chunk(_chunk_slices(dz_ref, _CH * c)) for c in range(_R // _CH)],
        axis=0)

    both = (dx_ref[...] <= t8x) & (dz_ref[...] <= t8z)
    c = jnp.sum(both.astype(jnp.float32))
    out_ref[...] = jnp.full((8, 128), (16.0 * _R - 2.0 * c) * (100.0 / (_N * _N)),
                            jnp.float32)


def kernel(X, Z):
    n, dx = X.shape
    _, dz = Z.shape
    out = pl.pallas_call(
        _body,
        grid=(n // _R,),
        in_specs=[
            pl.BlockSpec((_R, dx), lambda i: (i, 0)),
            pl.BlockSpec((n, dx), lambda i: (0, 0)),
            pl.BlockSpec((_R, dz), lambda i: (i, 0)),
            pl.BlockSpec((n, dz), lambda i: (0, 0)),
        ],
        out_specs=pl.BlockSpec((8, 128), lambda i: (i, 0)),
        out_shape=jax.ShapeDtypeStruct((8 * (n // _R), 128), jnp.float32),
        scratch_shapes=[
            pltpu.VMEM((_R, _N), jnp.float32),
            pltpu.VMEM((_R, _N), jnp.float32),
            pltpu.VMEM((1, _N), jnp.float32),
            pltpu.VMEM((1, _N), jnp.float32),
        ],
        compiler_params=pltpu.CompilerParams(
            dimension_semantics=("arbitrary",)),
    )(X, X, Z, Z)
    return jnp.sum(out[::8, 0])
